# Initial kernel scaffold; baseline (speedup 1.0000x reference)
import functools

import jax
import jax.numpy as jnp
from jax import lax
from jax.experimental import pallas as pl
from jax.experimental.pallas import tpu as pltpu
from jax.experimental.pallas import tpu_sc as plsc

_NUM_SEGMENTS = 256
_N = 100000
_D = 128
_L = 16
_NC = 2
_NS = 16
_NW = _NC * _NS
_SPW = _NUM_SEGMENTS // _NW
_NJ = _D // _L
_T = 500
_BPAD = 272


def _pool_body(x_hbm, bounds_hbm, out_hbm, bounds_v, buf, acc):
    wid = lax.axis_index("s") * _NC + lax.axis_index("c")

    pltpu.sync_copy(bounds_hbm.at[pl.ds(wid * _SPW, _L)], bounds_v)
    b = [bounds_v[i] for i in range(_SPW + 1)]

    zeros = jnp.zeros((_L,), jnp.float32)
    for k in range(_SPW):
        for j in range(_NJ):
            acc[k, pl.ds(j * _L, _L)] = zeros

    t_first = b[0] // _T
    t_last = (b[_SPW] - 1) // _T

    def tile_body(t, carry):
        base = t * _T
        pltpu.sync_copy(x_hbm.at[pl.ds(base, _T), :], buf)
        for k in range(_SPW):
            lo = jnp.maximum(b[k], base)
            hi = jnp.minimum(b[k + 1], base + _T)

            def row_body(r, c):
                rl = r - base
                return tuple(c[j] + buf[rl, pl.ds(j * _L, _L)]
                             for j in range(_NJ))

            init = tuple(jnp.zeros((_L,), jnp.float32) for _ in range(_NJ))
            csum = lax.fori_loop(lo, hi, row_body, init)
            for j in range(_NJ):
                acc[k, pl.ds(j * _L, _L)] = acc[k, pl.ds(j * _L, _L)] + csum[j]
        return carry

    lax.fori_loop(t_first, t_last + 1, tile_body, 0)

    for k in range(_SPW):
        cnt = jnp.maximum(b[k + 1] - b[k], 1).astype(jnp.float32)
        inv = 1.0 / jnp.full((_L,), cnt, jnp.float32)
        for j in range(_NJ):
            acc[k, pl.ds(j * _L, _L)] = acc[k, pl.ds(j * _L, _L)] * inv

    pltpu.sync_copy(acc, out_hbm.at[pl.ds(wid * _SPW, _SPW), :])


_pool = functools.partial(
    pl.kernel,
    out_type=jax.ShapeDtypeStruct((_NUM_SEGMENTS, _D), jnp.float32),
    mesh=plsc.VectorSubcoreMesh(core_axis_name="c", subcore_axis_name="s"),
    scratch_types=[
        pltpu.VMEM((_L,), jnp.int32),
        pltpu.VMEM((_T, _D), jnp.float32),
        pltpu.VMEM((_SPW, _D), jnp.float32),
    ],
)(_pool_body)


def kernel(input, node_segment):
    seg = node_segment.astype(jnp.int32)
    starts = jnp.searchsorted(
        seg, jnp.arange(_NUM_SEGMENTS, dtype=jnp.int32), side="left"
    ).astype(jnp.int32)
    bounds = jnp.concatenate(
        [starts, jnp.full((_BPAD - _NUM_SEGMENTS,), _N, jnp.int32)]
    )
    return _pool(input, bounds)


# speedup vs baseline: 4.9391x; 4.9391x over previous
import functools

import jax
import jax.numpy as jnp
from jax import lax
from jax.experimental import pallas as pl
from jax.experimental.pallas import tpu as pltpu
from jax.experimental.pallas import tpu_sc as plsc

_NUM_SEGMENTS = 256
_N = 100000
_D = 128
_L = 16
_NC = 2
_NS = 16
_NW = _NC * _NS
_SPW = _NUM_SEGMENTS // _NW
_NJ = _D // _L
_T = 400
_BPAD = 272


def _pool_body(x_hbm, bounds_hbm, out_hbm, bounds_v, buf, acc):
    wid = lax.axis_index("s") * _NC + lax.axis_index("c")

    pltpu.sync_copy(bounds_hbm.at[pl.ds(wid * _SPW, _L)], bounds_v)
    bvec = bounds_v[:]
    b = [bvec[i] for i in range(_SPW + 1)]

    zeros = jnp.zeros((_L,), jnp.float32)
    for k in range(_SPW):
        for j in range(_NJ):
            acc[k, pl.ds(j * _L, _L)] = zeros

    t_first = b[0] // _T
    t_last = (b[_SPW] - 1) // _T

    def tile_body(t, carry):
        base = t * _T
        pltpu.sync_copy(x_hbm.at[pl.ds(base, _T), :], buf)
        for k in range(_SPW):
            lo = jnp.maximum(b[k], base)
            hi = jnp.minimum(b[k + 1], base + _T)

            def row_body(r, c):
                rl = r - base
                return tuple(c[j] + buf[rl, pl.ds(j * _L, _L)]
                             for j in range(_NJ))

            init = tuple(jnp.zeros((_L,), jnp.float32) for _ in range(_NJ))
            csum = lax.fori_loop(lo, hi, row_body, init)
            for j in range(_NJ):
                acc[k, pl.ds(j * _L, _L)] = acc[k, pl.ds(j * _L, _L)] + csum[j]
        return carry

    lax.fori_loop(t_first, t_last + 1, tile_body, 0)

    for k in range(_SPW):
        cnt = jnp.maximum(b[k + 1] - b[k], 1).astype(jnp.float32)
        inv = 1.0 / jnp.full((_L,), cnt, jnp.float32)
        for j in range(_NJ):
            acc[k, pl.ds(j * _L, _L)] = acc[k, pl.ds(j * _L, _L)] * inv

    pltpu.sync_copy(acc, out_hbm.at[pl.ds(wid * _SPW, _SPW), :])


_pool = functools.partial(
    pl.kernel,
    out_type=jax.ShapeDtypeStruct((_NUM_SEGMENTS, _D), jnp.float32),
    mesh=plsc.VectorSubcoreMesh(core_axis_name="c", subcore_axis_name="s"),
    scratch_types=[
        pltpu.VMEM((_L,), jnp.int32),
        pltpu.VMEM((_T, _D), jnp.float32),
        pltpu.VMEM((_SPW, _D), jnp.float32),
    ],
)(_pool_body)


def kernel(input, node_segment):
    seg = node_segment.astype(jnp.int32)
    starts = jnp.searchsorted(
        seg, jnp.arange(_NUM_SEGMENTS, dtype=jnp.int32), side="left"
    ).astype(jnp.int32)
    bounds = jnp.concatenate(
        [starts, jnp.full((_BPAD - _NUM_SEGMENTS,), _N, jnp.int32)]
    )
    return _pool(input, bounds)
